# baseline (speedup 1.0000x reference)
import functools

import jax
import jax.numpy as jnp
from jax import lax
from jax.experimental import pallas as pl
from jax.experimental.pallas import tpu as pltpu


def _fc1_stats_kernel(x_ref, w1_ref, b1_ref, h_ref, stats_ref, acc_ref):
    k = pl.program_id(1)
    nk = pl.num_programs(1)
    xb = x_ref[...].astype(jnp.bfloat16)
    w1b = w1_ref[...].astype(jnp.bfloat16)
    p = lax.dot_general(w1b, xb, (((1,), (0,)), ((), ())),
                        preferred_element_type=jnp.float32)

    @pl.when(k == 0)
    def _init():
        acc_ref[...] = p

    @pl.when(k > 0)
    def _acc():
        acc_ref[...] += p

    @pl.when(k == nk - 1)
    def _fin():
        h = acc_ref[...] + b1_ref[...].T
        h_ref[...] = h.astype(jnp.bfloat16)
        ones = jnp.ones((h.shape[1], 1), jnp.float32)
        s1 = lax.dot_general(h, ones, (((1,), (0,)), ((), ())),
                             preferred_element_type=jnp.float32)
        s2 = lax.dot_general(h * h, ones, (((1,), (0,)), ((), ())),
                             preferred_element_type=jnp.float32)
        stats_ref[0] = jnp.concatenate([s1, s2], axis=1)


def _bn_relu_fc2_kernel(h_ref, stats_ref, gamma_ref, beta_ref,
                        w2_ref, b2_ref, o_ref, *, b_total, eps):
    st = jnp.sum(stats_ref[...], axis=0)
    ssum = st[:, 0:1]
    ssq = st[:, 1:2]
    inv_b = 1.0 / float(b_total)
    mean = ssum * inv_b
    var = jnp.maximum(ssq * inv_b - mean * mean, 0.0)
    s = gamma_ref[...].T * lax.rsqrt(var + eps)
    t = beta_ref[...].T - mean * s
    a = jnp.maximum(h_ref[...].astype(jnp.float32) * s + t, 0.0)
    w2b = w2_ref[...].astype(jnp.bfloat16)
    y = jnp.dot(w2b, a.astype(jnp.bfloat16),
                preferred_element_type=jnp.float32)
    o_ref[...] = y + b2_ref[...].T


def kernel(x, w1, b1, gamma, beta, w2, b2, *, eps=1e-5):
    B = x.shape[0]
    In = x.size // B
    H = w1.shape[0]
    C = w2.shape[0]

    xt = x.reshape(B, In).T
    tile_b = min(2048, B)
    nb = B // tile_b
    tile_b2 = min(4096, B)
    nb2 = B // tile_b2

    b1r = b1.reshape(1, H)
    gr = gamma.reshape(1, H)
    br = beta.reshape(1, H)
    b2r = b2.reshape(1, C)

    nk = 2
    bk = In // nk
    ht, stats = pl.pallas_call(
        _fc1_stats_kernel,
        out_shape=(jax.ShapeDtypeStruct((H, B), jnp.bfloat16),
                   jax.ShapeDtypeStruct((nb, H, 2), jnp.float32)),
        grid=(nb, nk),
        in_specs=[pl.BlockSpec((bk, tile_b), lambda i, k: (k, i)),
                  pl.BlockSpec((H, bk), lambda i, k: (0, k)),
                  pl.BlockSpec((1, H), lambda i, k: (0, 0))],
        out_specs=(pl.BlockSpec((H, tile_b), lambda i, k: (0, i)),
                   pl.BlockSpec((1, H, 2), lambda i, k: (i, 0, 0))),
        scratch_shapes=[pltpu.VMEM((H, tile_b), jnp.float32)],
        compiler_params=pltpu.CompilerParams(
            dimension_semantics=("parallel", "arbitrary")),
        cost_estimate=pl.CostEstimate(
            flops=2 * B * In * H,
            transcendentals=0,
            bytes_accessed=4 * B * In + 4 * In * H + 2 * B * H + 8 * nb * H),
    )(xt, w1, b1r)

    yt = pl.pallas_call(
        functools.partial(_bn_relu_fc2_kernel, b_total=B, eps=eps),
        out_shape=jax.ShapeDtypeStruct((C, B), x.dtype),
        grid=(nb2,),
        in_specs=[pl.BlockSpec((H, tile_b2), lambda i: (0, i)),
                  pl.BlockSpec((nb, H, 2), lambda i: (0, 0, 0)),
                  pl.BlockSpec((1, H), lambda i: (0, 0)),
                  pl.BlockSpec((1, H), lambda i: (0, 0)),
                  pl.BlockSpec((C, H), lambda i: (0, 0)),
                  pl.BlockSpec((1, C), lambda i: (0, 0))],
        out_specs=pl.BlockSpec((C, tile_b2), lambda i: (0, i)),
        compiler_params=pltpu.CompilerParams(
            dimension_semantics=("parallel",)),
        cost_estimate=pl.CostEstimate(
            flops=2 * B * H * C,
            transcendentals=H,
            bytes_accessed=2 * B * H + 4 * H * C + 4 * B * C + 8 * nb * H),
    )(ht, stats, gr, br, w2, b2r)
    return yt.T


# speedup vs baseline: 1.2547x; 1.2547x over previous
import functools

import jax
import jax.numpy as jnp
from jax import lax
from jax.experimental import pallas as pl
from jax.experimental.pallas import tpu as pltpu


def _fc1_stats_kernel(x_ref, w1_ref, b1_ref, h_ref, stats_ref):
    xb = x_ref[...].astype(jnp.bfloat16)
    w1b = w1_ref[...].astype(jnp.bfloat16)
    h = lax.dot_general(w1b, xb, (((1,), (0,)), ((), ())),
                        preferred_element_type=jnp.float32)
    h = h + b1_ref[...].T
    h_ref[...] = h.astype(jnp.bfloat16)
    ones = jnp.ones((h.shape[1], 1), jnp.float32)
    s1 = lax.dot_general(h, ones, (((1,), (0,)), ((), ())),
                         preferred_element_type=jnp.float32)
    s2 = lax.dot_general(h * h, ones, (((1,), (0,)), ((), ())),
                         preferred_element_type=jnp.float32)
    stats_ref[0] = jnp.concatenate([s1, s2], axis=1)


def _bn_relu_fc2_kernel(h_ref, stats_ref, gamma_ref, beta_ref,
                        w2_ref, b2_ref, o_ref, *, b_total, eps):
    st = jnp.sum(stats_ref[...], axis=0)
    ssum = st[:, 0:1]
    ssq = st[:, 1:2]
    inv_b = 1.0 / float(b_total)
    mean = ssum * inv_b
    var = jnp.maximum(ssq * inv_b - mean * mean, 0.0)
    s = gamma_ref[...].T * lax.rsqrt(var + eps)
    t = beta_ref[...].T - mean * s
    a = jnp.maximum(h_ref[...].astype(jnp.float32) * s + t, 0.0)
    w2b = w2_ref[...].astype(jnp.bfloat16)
    y = jnp.dot(w2b, a.astype(jnp.bfloat16),
                preferred_element_type=jnp.float32)
    o_ref[...] = y + b2_ref[...].T


def kernel(x, w1, b1, gamma, beta, w2, b2, *, eps=1e-5):
    B = x.shape[0]
    In = x.size // B
    H = w1.shape[0]
    C = w2.shape[0]

    xt = x.reshape(B, In).T
    tile_b = min(2048, B)
    nb = B // tile_b
    tile_b2 = min(2048, B)
    nb2 = B // tile_b2

    b1r = b1.reshape(1, H)
    gr = gamma.reshape(1, H)
    br = beta.reshape(1, H)
    b2r = b2.reshape(1, C)

    ht, stats = pl.pallas_call(
        _fc1_stats_kernel,
        out_shape=(jax.ShapeDtypeStruct((H, B), jnp.bfloat16),
                   jax.ShapeDtypeStruct((nb, H, 2), jnp.float32)),
        grid=(nb,),
        in_specs=[pl.BlockSpec((In, tile_b), lambda i: (0, i)),
                  pl.BlockSpec((H, In), lambda i: (0, 0)),
                  pl.BlockSpec((1, H), lambda i: (0, 0))],
        out_specs=(pl.BlockSpec((H, tile_b), lambda i: (0, i)),
                   pl.BlockSpec((1, H, 2), lambda i: (i, 0, 0))),
        compiler_params=pltpu.CompilerParams(
            dimension_semantics=("arbitrary",)),
        cost_estimate=pl.CostEstimate(
            flops=2 * B * In * H,
            transcendentals=0,
            bytes_accessed=4 * B * In + 4 * In * H + 2 * B * H + 8 * nb * H),
    )(xt, w1, b1r)

    yt = pl.pallas_call(
        functools.partial(_bn_relu_fc2_kernel, b_total=B, eps=eps),
        out_shape=jax.ShapeDtypeStruct((C, B), x.dtype),
        grid=(nb2,),
        in_specs=[pl.BlockSpec((H, tile_b2), lambda i: (0, i)),
                  pl.BlockSpec((nb, H, 2), lambda i: (0, 0, 0)),
                  pl.BlockSpec((1, H), lambda i: (0, 0)),
                  pl.BlockSpec((1, H), lambda i: (0, 0)),
                  pl.BlockSpec((C, H), lambda i: (0, 0)),
                  pl.BlockSpec((1, C), lambda i: (0, 0))],
        out_specs=pl.BlockSpec((C, tile_b2), lambda i: (0, i)),
        compiler_params=pltpu.CompilerParams(
            dimension_semantics=("parallel",)),
        cost_estimate=pl.CostEstimate(
            flops=2 * B * H * C,
            transcendentals=H,
            bytes_accessed=2 * B * H + 4 * H * C + 4 * B * C + 8 * nb * H),
    )(ht, stats, gr, br, w2, b2r)
    return yt.T


# speedup vs baseline: 1.3063x; 1.0412x over previous
import functools

import jax
import jax.numpy as jnp
from jax import lax
from jax.experimental import pallas as pl
from jax.experimental.pallas import tpu as pltpu


def _fused_kernel(x_ref, w1_ref, b1_ref, gamma_ref, beta_ref, w2_ref,
                  b2_ref, o_ref, h_scr, st_scr, *, nb, tile_b, b_total, eps):
    s = pl.program_id(0)

    @pl.when(s < nb)
    def _phase0():
        i = s
        xb = x_ref[...].astype(jnp.bfloat16)
        w1b = w1_ref[...].astype(jnp.bfloat16)
        h = lax.dot_general(w1b, xb, (((1,), (0,)), ((), ())),
                            preferred_element_type=jnp.float32)
        h = h + b1_ref[...].T
        h_scr[:, pl.ds(i * tile_b, tile_b)] = h.astype(jnp.bfloat16)
        ones = jnp.ones((tile_b, 1), jnp.float32)
        s1 = lax.dot_general(h, ones, (((1,), (0,)), ((), ())),
                             preferred_element_type=jnp.float32)
        s2 = lax.dot_general(h * h, ones, (((1,), (0,)), ((), ())),
                             preferred_element_type=jnp.float32)
        st = jnp.concatenate([s1, s2], axis=1)

        @pl.when(i == 0)
        def _init():
            st_scr[...] = st

        @pl.when(i > 0)
        def _acc():
            st_scr[...] += st

    @pl.when(s >= nb)
    def _phase1():
        i = s - nb
        st = st_scr[...]
        inv_b = 1.0 / float(b_total)
        mean = st[:, 0:1] * inv_b
        var = jnp.maximum(st[:, 1:2] * inv_b - mean * mean, 0.0)
        sc = gamma_ref[...].T * lax.rsqrt(var + eps)
        tc = beta_ref[...].T - mean * sc
        hb = h_scr[:, pl.ds(i * tile_b, tile_b)]
        a = jnp.maximum(hb.astype(jnp.float32) * sc + tc, 0.0)
        w2b = w2_ref[...].astype(jnp.bfloat16)
        y = jnp.dot(w2b, a.astype(jnp.bfloat16),
                    preferred_element_type=jnp.float32)
        o_ref[...] = y + b2_ref[...].T


def kernel(x, w1, b1, gamma, beta, w2, b2, *, eps=1e-5):
    B = x.shape[0]
    In = x.size // B
    H = w1.shape[0]
    C = w2.shape[0]

    xt = x.reshape(B, In).T
    tile_b = min(1024, B)
    nb = B // tile_b

    b1r = b1.reshape(1, H)
    gr = gamma.reshape(1, H)
    br = beta.reshape(1, H)
    b2r = b2.reshape(1, C)

    nbc = nb

    yt = pl.pallas_call(
        functools.partial(_fused_kernel, nb=nb, tile_b=tile_b,
                          b_total=B, eps=eps),
        out_shape=jax.ShapeDtypeStruct((C, B), x.dtype),
        grid=(2 * nb,),
        in_specs=[pl.BlockSpec((In, tile_b),
                               lambda s: (0, jnp.minimum(s, nbc - 1))),
                  pl.BlockSpec((H, In), lambda s: (0, 0)),
                  pl.BlockSpec((1, H), lambda s: (0, 0)),
                  pl.BlockSpec((1, H), lambda s: (0, 0)),
                  pl.BlockSpec((1, H), lambda s: (0, 0)),
                  pl.BlockSpec((C, H), lambda s: (0, 0)),
                  pl.BlockSpec((1, C), lambda s: (0, 0))],
        out_specs=pl.BlockSpec((C, tile_b),
                               lambda s: (0, jnp.maximum(s - nbc, 0))),
        scratch_shapes=[pltpu.VMEM((H, B), jnp.bfloat16),
                        pltpu.VMEM((H, 2), jnp.float32)],
        compiler_params=pltpu.CompilerParams(
            dimension_semantics=("arbitrary",)),
        cost_estimate=pl.CostEstimate(
            flops=2 * B * In * H + 2 * B * H * C,
            transcendentals=H,
            bytes_accessed=4 * B * In + 4 * In * H + 4 * H * C + 4 * B * C),
    )(xt, w1, b1r, gr, br, w2, b2r)
    return yt.T


# speedup vs baseline: 1.3884x; 1.0628x over previous
import functools

import jax
import jax.numpy as jnp
from jax import lax
from jax.experimental import pallas as pl
from jax.experimental.pallas import tpu as pltpu


def _fused_kernel(x_ref, w1_ref, b1_ref, gamma_ref, beta_ref, w2_ref,
                  b2_ref, o_ref, h_scr, st_scr, *, nb, tile_b, b_total, eps):
    s = pl.program_id(0)

    @pl.when(s < nb)
    def _phase0():
        i = s
        xb = x_ref[...].astype(jnp.bfloat16)
        w1b = w1_ref[...].astype(jnp.bfloat16)
        h = lax.dot_general(w1b, xb, (((1,), (0,)), ((), ())),
                            preferred_element_type=jnp.float32)
        h = h + b1_ref[...].T
        h_scr[:, pl.ds(i * tile_b, tile_b)] = h.astype(jnp.bfloat16)
        ones = jnp.ones((tile_b, 1), jnp.float32)
        s1 = lax.dot_general(h, ones, (((1,), (0,)), ((), ())),
                             preferred_element_type=jnp.float32)
        s2 = lax.dot_general(h * h, ones, (((1,), (0,)), ((), ())),
                             preferred_element_type=jnp.float32)
        st = jnp.concatenate([s1, s2], axis=1)

        @pl.when(i == 0)
        def _init():
            st_scr[...] = st

        @pl.when(i > 0)
        def _acc():
            st_scr[...] += st

    @pl.when(s >= nb)
    def _phase1():
        i = s - nb
        st = st_scr[...]
        inv_b = 1.0 / float(b_total)
        mean = st[:, 0:1] * inv_b
        var = jnp.maximum(st[:, 1:2] * inv_b - mean * mean, 0.0)
        sc = gamma_ref[...].T * lax.rsqrt(var + eps)
        tc = beta_ref[...].T - mean * sc
        hb = h_scr[:, pl.ds(i * tile_b, tile_b)]
        a = jnp.maximum(hb.astype(jnp.float32) * sc + tc, 0.0)
        w2b = w2_ref[...].astype(jnp.bfloat16)
        y = jnp.dot(w2b, a.astype(jnp.bfloat16),
                    preferred_element_type=jnp.float32)
        o_ref[...] = y + b2_ref[...].T


def kernel(x, w1, b1, gamma, beta, w2, b2, *, eps=1e-5):
    B = x.shape[0]
    In = x.size // B
    H = w1.shape[0]
    C = w2.shape[0]

    xt = x.reshape(B, In).T
    tile_b = min(2048, B)
    nb = B // tile_b

    b1r = b1.reshape(1, H)
    gr = gamma.reshape(1, H)
    br = beta.reshape(1, H)
    b2r = b2.reshape(1, C)

    nbc = nb

    yt = pl.pallas_call(
        functools.partial(_fused_kernel, nb=nb, tile_b=tile_b,
                          b_total=B, eps=eps),
        out_shape=jax.ShapeDtypeStruct((C, B), x.dtype),
        grid=(2 * nb,),
        in_specs=[pl.BlockSpec((In, tile_b),
                               lambda s: (0, jnp.minimum(s, nbc - 1))),
                  pl.BlockSpec((H, In), lambda s: (0, 0)),
                  pl.BlockSpec((1, H), lambda s: (0, 0)),
                  pl.BlockSpec((1, H), lambda s: (0, 0)),
                  pl.BlockSpec((1, H), lambda s: (0, 0)),
                  pl.BlockSpec((C, H), lambda s: (0, 0)),
                  pl.BlockSpec((1, C), lambda s: (0, 0))],
        out_specs=pl.BlockSpec((C, tile_b),
                               lambda s: (0, jnp.maximum(s - nbc, 0))),
        scratch_shapes=[pltpu.VMEM((H, B), jnp.bfloat16),
                        pltpu.VMEM((H, 2), jnp.float32)],
        compiler_params=pltpu.CompilerParams(
            dimension_semantics=("arbitrary",)),
        cost_estimate=pl.CostEstimate(
            flops=2 * B * In * H + 2 * B * H * C,
            transcendentals=H,
            bytes_accessed=4 * B * In + 4 * In * H + 4 * H * C + 4 * B * C),
    )(xt, w1, b1r, gr, br, w2, b2r)
    return yt.T
